# baseline (speedup 1.0000x reference)
import functools

import jax
import jax.numpy as jnp
from jax import lax
from jax.experimental import pallas as pl
from jax.experimental.pallas import tpu as pltpu
from jax.experimental.pallas import tpu_sc as plsc

N_NODES = 10000
D = 128
N_EDGES = 320000
EPS = 1e-5

NC = 2
NS = 16
DH = D // NC
EPW = N_EDGES // NS
K = 80
NCHUNK = EPW // K
W = 25
NWIN = NCHUNK // W
NBUF = 5
N_PAD = 10240
ROWS_PER_SUB = N_PAD // NS
XROWS_PER_SUB = N_NODES // NS
DEG_W = 16

_mesh = plsc.VectorSubcoreMesh(core_axis_name="c", subcore_axis_name="s")


@functools.partial(
    pl.kernel,
    mesh=_mesh,
    compiler_params=pltpu.CompilerParams(use_tc_tiling_on_sc=False),
    out_type=[
        jax.ShapeDtypeStruct((N_PAD, D), jnp.float32),
        jax.ShapeDtypeStruct((N_PAD, NC * DEG_W), jnp.float32),
    ],
    scratch_types=[
        pltpu.VMEM((2, W, K), jnp.int32),
        pltpu.VMEM((2, W, K), jnp.int32),
        pltpu.VMEM((K, DEG_W), jnp.float32),
        *[pltpu.VMEM((K, DH), jnp.float32) for _ in range(NBUF)],
        pltpu.VMEM_SHARED((N_NODES, DH), jnp.float32),
        pltpu.VMEM_SHARED((N_PAD, DH), jnp.float32),
        pltpu.VMEM_SHARED((N_PAD, DEG_W), jnp.float32),
        *[pltpu.SemaphoreType.DMA for _ in range(NBUF)],
        *[pltpu.SemaphoreType.DMA for _ in range(4)],
    ],
)
def _sc_aggregate(x_hbm, ei_hbm, zacc_hbm, zdeg_hbm,
                  acc_out_hbm, deg_out_hbm,
                  src_v, dst_v, ones_v, *rest):
    bufs = rest[:NBUF]
    x_s = rest[NBUF]
    acc_s = rest[NBUF + 1]
    deg_s = rest[NBUF + 2]
    sems = rest[NBUF + 3:NBUF + 3 + NBUF]
    isems = rest[NBUF + 3 + NBUF:]

    c = lax.axis_index("c")
    s = lax.axis_index("s")

    def idx_copies(w, slot):
        return (
            pltpu.make_async_copy(ei_hbm.at[0, s, pl.ds(w * W, W)],
                                  src_v.at[slot], isems[0 + 2 * (slot % 2)]),
            pltpu.make_async_copy(ei_hbm.at[1, s, pl.ds(w * W, W)],
                                  dst_v.at[slot], isems[1 + 2 * (slot % 2)]),
        )

    def idx_start(w, slot):
        for cp in idx_copies(w, slot):
            cp.start()

    def idx_wait(w, slot):
        for cp in idx_copies(w, slot):
            cp.wait()

    idx_start(0, 0)

    one16 = jnp.ones((16,), jnp.float32)
    for i in range(K):
        ones_v[i, :] = one16

    xrow0 = s * XROWS_PER_SUB
    pltpu.sync_copy(x_hbm.at[pl.ds(xrow0, XROWS_PER_SUB), pl.ds(c * DH, DH)],
                    x_s.at[pl.ds(xrow0, XROWS_PER_SUB)])
    row0 = s * ROWS_PER_SUB
    pltpu.sync_copy(zacc_hbm.at[pl.ds(row0, ROWS_PER_SUB)],
                    acc_s.at[pl.ds(row0, ROWS_PER_SUB)])
    pltpu.sync_copy(zdeg_hbm.at[pl.ds(row0, ROWS_PER_SUB)],
                    deg_s.at[pl.ds(row0, ROWS_PER_SUB)])
    plsc.subcore_barrier()

    idx_wait(0, 0)

    def gather_start(slot, row, b):
        pltpu.make_async_copy(x_s.at[src_v.at[slot, row]], bufs[b],
                              sems[b]).start()

    def gather_wait(slot, row, b):
        pltpu.make_async_copy(x_s.at[src_v.at[slot, row]], bufs[b],
                              sems[b]).wait()

    def scatter(j, slot, row, b):
        pltpu.sync_copy(bufs[b], acc_s.at[dst_v.at[slot, row]], add=True)
        do_deg = jnp.logical_xor(j < NCHUNK // 2, c == 1)

        @pl.when(do_deg)
        def _():
            pltpu.sync_copy(ones_v, deg_s.at[dst_v.at[slot, row]], add=True)

    for b in range(NBUF):
        gather_start(0, b, b)

    def do_window(base, slot, next_w, last):
        nslot = 1 - slot
        if not last:
            idx_start(next_w, nslot)
        for k in range(W):
            j = base + k
            gather_wait(slot, k, k % NBUF)
            scatter(j, slot, k, k % NBUF)
            if last:
                if k < W - NBUF:
                    gather_start(slot, k + NBUF, k % NBUF)
            else:
                if k == W - NBUF:
                    idx_wait(next_w, nslot)
                la, ls = k + NBUF, slot
                if la >= W:
                    la, ls = la - W, nslot
                gather_start(ls, la, k % NBUF)

    def pair_body(p, carry):
        w0 = 2 * p
        do_window(w0 * W, 0, w0 + 1, False)
        do_window((w0 + 1) * W, 1, w0 + 2, False)
        return carry

    lax.fori_loop(0, NWIN // 2 - 1, pair_body, 0)

    do_window((NWIN - 2) * W, 0, NWIN - 1, False)
    do_window((NWIN - 1) * W, 1, 0, True)

    plsc.subcore_barrier()

    pltpu.sync_copy(acc_s.at[pl.ds(row0, ROWS_PER_SUB)],
                    acc_out_hbm.at[pl.ds(row0, ROWS_PER_SUB),
                                   pl.ds(c * DH, DH)])
    pltpu.sync_copy(deg_s.at[pl.ds(row0, ROWS_PER_SUB)],
                    deg_out_hbm.at[pl.ds(row0, ROWS_PER_SUB),
                                   pl.ds(c * DEG_W, DEG_W)])


def _dense_body(p_ref, dp_ref, w_ref, b_ref, g_ref, be_ref, al_ref, o_ref):
    acc = p_ref[:N_NODES]
    deg = dp_ref[:N_NODES, 0:1] + dp_ref[:N_NODES, DEG_W:DEG_W + 1]
    agg = acc / jnp.maximum(deg, 1.0)
    h = jnp.dot(agg, w_ref[...], preferred_element_type=jnp.float32)
    h = h + b_ref[...]
    mean = jnp.mean(h, axis=0, keepdims=True)
    h_c = h - al_ref[...] * mean
    var = jnp.mean(h_c * h_c, axis=0, keepdims=True)
    out = g_ref[...] * (h_c * lax.rsqrt(var + EPS)) + be_ref[...]
    o_ref[...] = jnp.maximum(out, 0.0)


_dense = pl.pallas_call(
    _dense_body,
    out_shape=jax.ShapeDtypeStruct((N_NODES, D), jnp.float32),
)


def kernel(x, edge_index, W_mat, b, gamma, beta, alpha):
    ei = edge_index.astype(jnp.int32).reshape(2, NS, NCHUNK, K)
    zacc = jnp.zeros((N_PAD, DH), jnp.float32)
    zdeg = jnp.zeros((N_PAD, DEG_W), jnp.float32)
    acc_p, deg_p = _sc_aggregate(x, ei, zacc, zdeg)
    return _dense(acc_p, deg_p, W_mat,
                  b.reshape(1, D), gamma.reshape(1, D),
                  beta.reshape(1, D), alpha.reshape(1, D))
